# Initial kernel scaffold; baseline (speedup 1.0000x reference)
import jax
import jax.numpy as jnp
from jax.experimental import pallas as pl


def kernel(node_features, edge_weight, edge_index, W1, b1, W2, b2, W3, b3, W4, b4, W5, b5):
    raise NotImplementedError("write your pallas kernel here")


# speedup vs baseline: 4.7835x; 4.7835x over previous
import functools

import jax
import jax.numpy as jnp
from jax import lax
from jax.experimental import pallas as pl
from jax.experimental.pallas import tpu as pltpu
from jax.experimental.pallas import tpu_sc as plsc

N_NODES = 10000
N_EDGES = 320000
D_FEAT = 128
F = 16

NC = 2
NS = 16
NW = NC * NS
EPW = N_EDGES // NW
PIECE = 80
PROW = PIECE // 8
NPIECE = EPW // PIECE
NCHUNK = N_EDGES // PIECE
NPAD = 10240
RPW = NPAD // NS


@functools.cache
def _mesh():
    return plsc.VectorSubcoreMesh(
        core_axis_name="c", subcore_axis_name="s", num_cores=NC, num_subcores=NS)


def _pq_body(x_ref, w_ref, b_ref, t_ref):
    x = x_ref[...]
    t_ref[...] = (jnp.dot(x, w_ref[...], preferred_element_type=jnp.float32)
                  + b_ref[...])


def _stage_pq(x, w1ab, b1pad):
    return pl.pallas_call(
        _pq_body,
        out_shape=jax.ShapeDtypeStruct((NPAD, 128), jnp.float32),
    )(x, w1ab, b1pad)


def _gather_body(t_hbm, src_hbm, dst_hbm, ew_hbm, c1_hbm, s_hbm,
                 idx_s, idx_d, ew_v, c1_v, rows_s, rows_d, s_v,
                 cur_s, cur_d, sem):
    c = lax.axis_index("c")
    s = lax.axis_index("s")
    wid = s * NC + c
    pltpu.sync_copy(src_hbm.at[wid], idx_s)
    pltpu.sync_copy(dst_hbm.at[wid], idx_d)
    pltpu.sync_copy(ew_hbm.at[wid], ew_v)
    pltpu.sync_copy(c1_hbm, c1_v)

    c1v = c1_v[0, :]

    def body(j, carry):
        for g in range(PIECE // 16):
            cur_s[pl.ds(16 * g, 16)] = idx_s[j, pl.ds(16 * g, 16)]
            cur_d[pl.ds(16 * g, 16)] = idx_d[j, pl.ds(16 * g, 16)]
        cp = pltpu.async_copy(t_hbm.at[cur_s], rows_s, sem)
        cq = pltpu.async_copy(t_hbm.at[cur_d], rows_d, sem)
        cp.wait()
        cq.wait()

        def pack_group(g, carry2):
            wv = ew_v[j, pl.ds(g * 16, 16)]
            for k in range(16):
                i = g * 16 + k
                r = 2 * g + (k // 8)
                s_v[r, pl.ds(16 * (k % 8), 16)] = (
                    rows_s[i, pl.ds(0, 16)] + rows_d[i, pl.ds(16, 16)]
                    + wv[k] * c1v)
            return carry2
        lax.fori_loop(0, PIECE // 16, pack_group, 0)
        pltpu.sync_copy(s_v, s_hbm.at[wid * NPIECE + j])
        return carry
    lax.fori_loop(0, NPIECE, body, 0)


@functools.cache
def _stage_gather():
    return pl.kernel(
        _gather_body,
        out_type=jax.ShapeDtypeStruct((NCHUNK, PROW, 128), jnp.float32),
        mesh=_mesh(),
        scratch_types=[
            pltpu.VMEM((NPIECE, PIECE), jnp.int32),
            pltpu.VMEM((NPIECE, PIECE), jnp.int32),
            pltpu.VMEM((NPIECE, PIECE), jnp.float32),
            pltpu.VMEM((1, F), jnp.float32),
            pltpu.VMEM((PIECE, 128), jnp.float32),
            pltpu.VMEM((PIECE, 128), jnp.float32),
            pltpu.VMEM((PROW, 128), jnp.float32),
            pltpu.VMEM((PIECE,), jnp.int32),
            pltpu.VMEM((PIECE,), jnp.int32),
            pltpu.SemaphoreType.DMA,
        ],
    )


_CBLK = 160


def _edge_mlp_body(s_ref, w2_ref, b2_ref, w3_ref, b3_ref, e_ref):
    h1 = jnp.maximum(s_ref[...].reshape(_CBLK * PROW, 128), 0.0)
    h2 = jnp.dot(h1, w2_ref[...], preferred_element_type=jnp.float32)
    h2 = jnp.maximum(h2 + b2_ref[...], 0.0)
    z = jnp.dot(h2, w3_ref[...], preferred_element_type=jnp.float32)
    e_ref[...] = jax.nn.sigmoid(z + b3_ref[...]).reshape(_CBLK, PROW, 128)


def _stage_edge_mlp(S, W2bd, b2r, W3bd, b3r):
    nblk = NCHUNK // _CBLK
    full = lambda i: (0, 0)
    return pl.pallas_call(
        _edge_mlp_body,
        grid=(nblk,),
        in_specs=[
            pl.BlockSpec((_CBLK, PROW, 128), lambda i: (i, 0, 0)),
            pl.BlockSpec((128, 128), full),
            pl.BlockSpec((1, 128), full),
            pl.BlockSpec((128, 128), full),
            pl.BlockSpec((1, 128), full),
        ],
        out_specs=pl.BlockSpec((_CBLK, PROW, 128), lambda i: (i, 0, 0)),
        out_shape=jax.ShapeDtypeStruct((NCHUNK, PROW, 128), jnp.float32),
    )(S, W2bd, b2r, W3bd, b3r)


def _scatter_body(e_hbm, src_hbm, dst_hbm, zero_hbm, acc_hbm,
                  e_c, pos_v, neg_v, cur_s, cur_d, acc_sp):
    c = lax.axis_index("c")
    s = lax.axis_index("s")
    wid = s * NC + c

    pltpu.sync_copy(zero_hbm.at[pl.ds(s * RPW, RPW)],
                    acc_sp.at[pl.ds(s * RPW, RPW)])
    zv = jnp.zeros((16,), jnp.float32)
    for i in range(PIECE):
        for g in range(8):
            pos_v[i, pl.ds(16 * g, 16)] = zv
            neg_v[i, pl.ds(16 * g, 16)] = zv
    plsc.subcore_barrier()

    def body(j, carry):
        chunk = wid * NPIECE + j
        pltpu.sync_copy(e_hbm.at[chunk], e_c)
        pltpu.sync_copy(src_hbm.at[pl.ds(chunk * PIECE, PIECE)], cur_s)
        pltpu.sync_copy(dst_hbm.at[pl.ds(chunk * PIECE, PIECE)], cur_d)

        def unpack_row(r, carry2):
            for k in range(8):
                i = 8 * r + k
                v = e_c[r, pl.ds(16 * k, 16)]
                pos_v[i, pl.ds(0, 16)] = v
                neg_v[i, pl.ds(0, 16)] = -v
            return carry2
        lax.fori_loop(0, PROW, unpack_row, 0)
        pltpu.sync_copy(pos_v, acc_sp.at[cur_d], add=True)
        pltpu.sync_copy(neg_v, acc_sp.at[cur_s], add=True)
        return carry
    lax.fori_loop(0, NPIECE, body, 0)

    plsc.subcore_barrier()
    pltpu.sync_copy(acc_sp.at[pl.ds(s * RPW, RPW)],
                    acc_hbm.at[c, pl.ds(s * RPW, RPW)])


@functools.cache
def _stage_scatter():
    return pl.kernel(
        _scatter_body,
        out_type=jax.ShapeDtypeStruct((NC, NPAD, 128), jnp.float32),
        mesh=_mesh(),
        scratch_types=[
            pltpu.VMEM((PROW, 128), jnp.float32),
            pltpu.VMEM((PIECE, 128), jnp.float32),
            pltpu.VMEM((PIECE, 128), jnp.float32),
            pltpu.VMEM((PIECE,), jnp.int32),
            pltpu.VMEM((PIECE,), jnp.int32),
            pltpu.VMEM_SHARED((NPAD, 128), jnp.float32),
        ],
    )


def _node_mlp_body(acc_ref, w4_ref, b4_ref, w5_ref, b5_ref, o_ref):
    a = acc_ref[0, :, :F] + acc_ref[1, :, :F]
    h = jnp.dot(a, w4_ref[...], preferred_element_type=jnp.float32)
    h = jnp.maximum(h + b4_ref[...], 0.0)
    z = jnp.dot(h, w5_ref[...], preferred_element_type=jnp.float32)
    o_ref[...] = jax.nn.sigmoid(z + b5_ref[...])


def _stage_node_mlp(acc, W4, b4, W5, b5):
    return pl.pallas_call(
        _node_mlp_body,
        out_shape=jax.ShapeDtypeStruct((NPAD, 1), jnp.float32),
    )(acc, W4, b4, W5, b5)


def kernel(node_features, edge_weight, edge_index, W1, b1, W2, b2, W3, b3, W4, b4, W5, b5):
    x = jnp.pad(node_features[0], ((0, NPAD - N_NODES), (0, 0)))
    src = edge_index[0].astype(jnp.int32).reshape(NW, NPIECE, PIECE)
    dst = edge_index[1].astype(jnp.int32).reshape(NW, NPIECE, PIECE)
    ew = edge_weight[0].reshape(NW, NPIECE, PIECE)

    w1a = W1[:D_FEAT]
    w1b = W1[D_FEAT:2 * D_FEAT]
    c1 = W1[2 * D_FEAT].reshape(1, F)

    eye8 = jnp.eye(8, dtype=jnp.float32)
    W2bd = jnp.kron(eye8, W2)
    W3bd = jnp.kron(eye8, W3)
    b2r = jnp.tile(b2, 8).reshape(1, 128)
    b3r = jnp.tile(b3, 8).reshape(1, 128)

    w1ab = jnp.concatenate(
        [w1a, w1b, jnp.zeros((D_FEAT, 128 - 2 * F), jnp.float32)], axis=1)
    b1pad = jnp.concatenate(
        [jnp.zeros((1, F), jnp.float32), b1.reshape(1, F),
         jnp.zeros((1, 128 - 2 * F), jnp.float32)], axis=1)
    T = _stage_pq(x, w1ab, b1pad)
    S = _stage_gather()(T, src, dst, ew, c1)
    e = _stage_edge_mlp(S, W2bd, b2r, W3bd, b3r)
    zeros = jnp.zeros((NPAD, 128), jnp.float32)
    acc = _stage_scatter()(e, src.reshape(-1), dst.reshape(-1), zeros)
    out = _stage_node_mlp(acc, W4, b4.reshape(1, 12), W5, b5.reshape(1, 1))
    return out[:N_NODES].reshape(1, N_NODES, 1)


# speedup vs baseline: 5.1679x; 1.0804x over previous
import functools

import jax
import jax.numpy as jnp
from jax import lax
from jax.experimental import pallas as pl
from jax.experimental.pallas import tpu as pltpu
from jax.experimental.pallas import tpu_sc as plsc

N_NODES = 10000
N_EDGES = 320000
D_FEAT = 128
F = 16

NC = 2
NS = 16
NW = NC * NS
EPW = N_EDGES // NW
PIECE = 80
PROW = PIECE // 8
NPIECE = EPW // PIECE
NCHUNK = N_EDGES // PIECE
NPAD = 10240
RPW = NPAD // NS


@functools.cache
def _mesh():
    return plsc.VectorSubcoreMesh(
        core_axis_name="c", subcore_axis_name="s", num_cores=NC, num_subcores=NS)


def _pq_body(x_ref, w_ref, b_ref, t_ref):
    x = x_ref[...]
    t_ref[...] = (jnp.dot(x, w_ref[...], preferred_element_type=jnp.float32)
                  + b_ref[...])


def _stage_pq(x, w1ab, b1pad):
    return pl.pallas_call(
        _pq_body,
        out_shape=jax.ShapeDtypeStruct((NPAD, 128), jnp.float32),
    )(x, w1ab, b1pad)


def _gather_body(t_hbm, src_hbm, dst_hbm, ew_hbm, c1_hbm, s_hbm,
                 idx_s, idx_d, ew_v, c1_v, rows_s, rows_d, rows_s1, rows_d1,
                 s_v, cur_s, cur_d, cur_s1, cur_d1, sem, sem1):
    c = lax.axis_index("c")
    s = lax.axis_index("s")
    wid = s * NC + c
    pltpu.sync_copy(src_hbm.at[wid], idx_s)
    pltpu.sync_copy(dst_hbm.at[wid], idx_d)
    pltpu.sync_copy(ew_hbm.at[wid], ew_v)
    pltpu.sync_copy(c1_hbm, c1_v)

    c1v = c1_v[0, :]

    def fill_cur(j, cs, cd):
        for g in range(PIECE // 16):
            cs[pl.ds(16 * g, 16)] = idx_s[j, pl.ds(16 * g, 16)]
            cd[pl.ds(16 * g, 16)] = idx_d[j, pl.ds(16 * g, 16)]

    def pack_store(j, rs, rd):
        def pack_group(g, carry2):
            wv = ew_v[j, pl.ds(g * 16, 16)]
            for k in range(16):
                i = g * 16 + k
                r = 2 * g + (k // 8)
                s_v[r, pl.ds(16 * (k % 8), 16)] = (
                    rs[i, pl.ds(0, 16)] + rd[i, pl.ds(16, 16)]
                    + wv[k] * c1v)
            return carry2
        lax.fori_loop(0, PIECE // 16, pack_group, 0)
        pltpu.sync_copy(s_v, s_hbm.at[wid * NPIECE + j])

    def body(t, carry):
        j0 = 2 * t
        j1 = 2 * t + 1
        fill_cur(j0, cur_s, cur_d)
        cp = pltpu.async_copy(t_hbm.at[cur_s], rows_s, sem)
        cq = pltpu.async_copy(t_hbm.at[cur_d], rows_d, sem)
        fill_cur(j1, cur_s1, cur_d1)
        cp1 = pltpu.async_copy(t_hbm.at[cur_s1], rows_s1, sem1)
        cq1 = pltpu.async_copy(t_hbm.at[cur_d1], rows_d1, sem1)
        cp.wait()
        cq.wait()
        pack_store(j0, rows_s, rows_d)
        cp1.wait()
        cq1.wait()
        pack_store(j1, rows_s1, rows_d1)
        return carry
    lax.fori_loop(0, NPIECE // 2, body, 0)

    jt = NPIECE - 1
    fill_cur(jt, cur_s, cur_d)
    cp = pltpu.async_copy(t_hbm.at[cur_s], rows_s, sem)
    cq = pltpu.async_copy(t_hbm.at[cur_d], rows_d, sem)
    cp.wait()
    cq.wait()
    pack_store(jt, rows_s, rows_d)


@functools.cache
def _stage_gather():
    return pl.kernel(
        _gather_body,
        out_type=jax.ShapeDtypeStruct((NCHUNK, PROW, 128), jnp.float32),
        mesh=_mesh(),
        scratch_types=[
            pltpu.VMEM((NPIECE, PIECE), jnp.int32),
            pltpu.VMEM((NPIECE, PIECE), jnp.int32),
            pltpu.VMEM((NPIECE, PIECE), jnp.float32),
            pltpu.VMEM((1, F), jnp.float32),
            pltpu.VMEM((PIECE, 128), jnp.float32),
            pltpu.VMEM((PIECE, 128), jnp.float32),
            pltpu.VMEM((PIECE, 128), jnp.float32),
            pltpu.VMEM((PIECE, 128), jnp.float32),
            pltpu.VMEM((PROW, 128), jnp.float32),
            pltpu.VMEM((PIECE,), jnp.int32),
            pltpu.VMEM((PIECE,), jnp.int32),
            pltpu.VMEM((PIECE,), jnp.int32),
            pltpu.VMEM((PIECE,), jnp.int32),
            pltpu.SemaphoreType.DMA,
            pltpu.SemaphoreType.DMA,
        ],
    )


_CBLK = 160


def _edge_mlp_body(s_ref, w2_ref, b2_ref, w3_ref, b3_ref, e_ref):
    h1 = jnp.maximum(s_ref[...].reshape(_CBLK * PROW, 128), 0.0)
    h2 = jnp.dot(h1, w2_ref[...], preferred_element_type=jnp.float32)
    h2 = jnp.maximum(h2 + b2_ref[...], 0.0)
    z = jnp.dot(h2, w3_ref[...], preferred_element_type=jnp.float32)
    e_ref[...] = jax.nn.sigmoid(z + b3_ref[...]).reshape(_CBLK, PROW, 128)


def _stage_edge_mlp(S, W2bd, b2r, W3bd, b3r):
    nblk = NCHUNK // _CBLK
    full = lambda i: (0, 0)
    return pl.pallas_call(
        _edge_mlp_body,
        grid=(nblk,),
        in_specs=[
            pl.BlockSpec((_CBLK, PROW, 128), lambda i: (i, 0, 0)),
            pl.BlockSpec((128, 128), full),
            pl.BlockSpec((1, 128), full),
            pl.BlockSpec((128, 128), full),
            pl.BlockSpec((1, 128), full),
        ],
        out_specs=pl.BlockSpec((_CBLK, PROW, 128), lambda i: (i, 0, 0)),
        out_shape=jax.ShapeDtypeStruct((NCHUNK, PROW, 128), jnp.float32),
    )(S, W2bd, b2r, W3bd, b3r)


def _scatter_body(e_hbm, src_hbm, dst_hbm, zero_hbm, acc_hbm,
                  e_c, pos_v, neg_v, cur_s, cur_d, acc_sp):
    c = lax.axis_index("c")
    s = lax.axis_index("s")
    wid = s * NC + c

    pltpu.sync_copy(zero_hbm.at[pl.ds(s * RPW, RPW)],
                    acc_sp.at[pl.ds(s * RPW, RPW)])
    zv = jnp.zeros((16,), jnp.float32)
    for i in range(PIECE):
        for g in range(8):
            pos_v[i, pl.ds(16 * g, 16)] = zv
            neg_v[i, pl.ds(16 * g, 16)] = zv
    plsc.subcore_barrier()

    def body(j, carry):
        chunk = wid * NPIECE + j
        pltpu.sync_copy(e_hbm.at[chunk], e_c)
        pltpu.sync_copy(src_hbm.at[pl.ds(chunk * PIECE, PIECE)], cur_s)
        pltpu.sync_copy(dst_hbm.at[pl.ds(chunk * PIECE, PIECE)], cur_d)

        def unpack_row(r, carry2):
            for k in range(8):
                i = 8 * r + k
                v = e_c[r, pl.ds(16 * k, 16)]
                pos_v[i, pl.ds(0, 16)] = v
                neg_v[i, pl.ds(0, 16)] = -v
            return carry2
        lax.fori_loop(0, PROW, unpack_row, 0)
        pltpu.sync_copy(pos_v, acc_sp.at[cur_d], add=True)
        pltpu.sync_copy(neg_v, acc_sp.at[cur_s], add=True)
        return carry
    lax.fori_loop(0, NPIECE, body, 0)

    plsc.subcore_barrier()
    pltpu.sync_copy(acc_sp.at[pl.ds(s * RPW, RPW)],
                    acc_hbm.at[c, pl.ds(s * RPW, RPW)])


@functools.cache
def _stage_scatter():
    return pl.kernel(
        _scatter_body,
        out_type=jax.ShapeDtypeStruct((NC, NPAD, 128), jnp.float32),
        mesh=_mesh(),
        scratch_types=[
            pltpu.VMEM((PROW, 128), jnp.float32),
            pltpu.VMEM((PIECE, 128), jnp.float32),
            pltpu.VMEM((PIECE, 128), jnp.float32),
            pltpu.VMEM((PIECE,), jnp.int32),
            pltpu.VMEM((PIECE,), jnp.int32),
            pltpu.VMEM_SHARED((NPAD, 128), jnp.float32),
        ],
    )


def _node_mlp_body(acc_ref, w4_ref, b4_ref, w5_ref, b5_ref, o_ref):
    a = acc_ref[0, :, :F] + acc_ref[1, :, :F]
    h = jnp.dot(a, w4_ref[...], preferred_element_type=jnp.float32)
    h = jnp.maximum(h + b4_ref[...], 0.0)
    z = jnp.dot(h, w5_ref[...], preferred_element_type=jnp.float32)
    o_ref[...] = jax.nn.sigmoid(z + b5_ref[...])


def _stage_node_mlp(acc, W4, b4, W5, b5):
    return pl.pallas_call(
        _node_mlp_body,
        out_shape=jax.ShapeDtypeStruct((NPAD, 1), jnp.float32),
    )(acc, W4, b4, W5, b5)


def kernel(node_features, edge_weight, edge_index, W1, b1, W2, b2, W3, b3, W4, b4, W5, b5):
    x = jnp.pad(node_features[0], ((0, NPAD - N_NODES), (0, 0)))
    src = edge_index[0].astype(jnp.int32).reshape(NW, NPIECE, PIECE)
    dst = edge_index[1].astype(jnp.int32).reshape(NW, NPIECE, PIECE)
    ew = edge_weight[0].reshape(NW, NPIECE, PIECE)

    w1a = W1[:D_FEAT]
    w1b = W1[D_FEAT:2 * D_FEAT]
    c1 = W1[2 * D_FEAT].reshape(1, F)

    eye8 = jnp.eye(8, dtype=jnp.float32)
    W2bd = jnp.kron(eye8, W2)
    W3bd = jnp.kron(eye8, W3)
    b2r = jnp.tile(b2, 8).reshape(1, 128)
    b3r = jnp.tile(b3, 8).reshape(1, 128)

    w1ab = jnp.concatenate(
        [w1a, w1b, jnp.zeros((D_FEAT, 128 - 2 * F), jnp.float32)], axis=1)
    b1pad = jnp.concatenate(
        [jnp.zeros((1, F), jnp.float32), b1.reshape(1, F),
         jnp.zeros((1, 128 - 2 * F), jnp.float32)], axis=1)
    T = _stage_pq(x, w1ab, b1pad)
    S = _stage_gather()(T, src, dst, ew, c1)
    e = _stage_edge_mlp(S, W2bd, b2r, W3bd, b3r)
    zeros = jnp.zeros((NPAD, 128), jnp.float32)
    acc = _stage_scatter()(e, src.reshape(-1), dst.reshape(-1), zeros)
    out = _stage_node_mlp(acc, W4, b4.reshape(1, 12), W5, b5.reshape(1, 1))
    return out[:N_NODES].reshape(1, N_NODES, 1)
